# Initial kernel scaffold; baseline (speedup 1.0000x reference)
import jax
import jax.numpy as jnp
from jax.experimental import pallas as pl


def kernel(boxes, scores, labels):
    raise NotImplementedError("write your pallas kernel here")


# speedup vs baseline: 11.7183x; 11.7183x over previous
import functools

import jax
import jax.numpy as jnp
from jax import lax
from jax.experimental import pallas as pl
from jax.experimental.pallas import tpu as pltpu
from jax.experimental.pallas import tpu_sc as plsc

N = 20000
NUM_CLASSES = 1203
IOU_THRESHOLD = 0.5
MIN_CONFIDENCE = 0.2

NC = 2
NS = 16
NW = NC * NS
CAP = 2048
NCHUNK = CAP // 128
CAPP = CAP + 16
PTOT = N + CAP
DUMP = N
STARTS_PAD = 1232
CB_PAD = 48


def _sread(ref, i):
    return ref[pl.ds(i, 16)][0]


def _nms_body(x1h, y1h, x2h, y2h, sch, oih, sth, cbh, outh,
              vx1, vy1, vx2, vy2, vsc, voi, var, vsup, vidx, vval,
              vstarts, vcb, sem):
    c = lax.axis_index("c")
    s = lax.axis_index("s")
    w = s * NC + c

    pltpu.sync_copy(sth, vstarts)
    pltpu.sync_copy(cbh, vcb)

    c_lo = _sread(vcb, w)
    c_hi = _sread(vcb, w + 1)
    s_w = _sread(vstarts, c_lo)
    e_w = _sread(vstarts, c_hi)
    s_al = (s_w // 16) * 16
    e_loc = jnp.minimum(e_w - s_al, CAP)

    for hbm, vm in ((x1h, vx1), (y1h, vy1), (x2h, vx2), (y2h, vy2),
                    (sch, vsc), (oih, voi)):
        pltpu.sync_copy(hbm.at[pl.ds(s_al, CAP)], vm.at[pl.ds(0, CAP)])

    lane = lax.iota(jnp.int32, 16)

    def init_chunk(k, _):
        sl = pl.ds(k * 16, 16)
        x1 = vx1[sl]
        y1 = vy1[sl]
        x2 = vx2[sl]
        y2 = vy2[sl]
        var[sl] = (x2 - x1) * (y2 - y1)
        vsup[sl] = jnp.zeros((16,), jnp.float32)
        gpos = s_al + k * 16 + lane
        owned = (gpos >= s_w) & (gpos < e_w)
        midx = jnp.where(owned, voi[sl], DUMP + w)
        vidx[k // 8, pl.ds((k % 8) * 16, 16)] = midx
        return 0

    lax.fori_loop(0, CAP // 16, init_chunk, 0)

    def class_body(cc, _):
        ce = jnp.minimum(_sread(vstarts, cc + 1) - s_al, CAP)
        ckn = (ce + 15) // 16

        def i_body(i, _):
            supi = _sread(vsup, i)
            sci = _sread(vsc, i)

            @pl.when((supi < 0.5) & (sci > MIN_CONFIDENCE))
            def _():
                bx1 = jnp.full((16,), _sread(vx1, i))
                by1 = jnp.full((16,), _sread(vy1, i))
                bx2 = jnp.full((16,), _sread(vx2, i))
                by2 = jnp.full((16,), _sread(vy2, i))
                ai = jnp.full((16,), _sread(var, i))

                def j_body(ck, _):
                    sl = pl.ds(ck * 16, 16)
                    jv = ck * 16 + lane
                    ix1 = jnp.maximum(bx1, vx1[sl])
                    iy1 = jnp.maximum(by1, vy1[sl])
                    ix2 = jnp.minimum(bx2, vx2[sl])
                    iy2 = jnp.minimum(by2, vy2[sl])
                    inter = jnp.maximum(ix2 - ix1, 0.0) * jnp.maximum(iy2 - iy1, 0.0)
                    iou = inter / (ai + var[sl] - inter + 1e-6)
                    hit = (iou > IOU_THRESHOLD) & (jv > i) & (jv < ce)
                    vsup[sl] = jnp.where(hit, 1.0, vsup[sl])
                    return 0

                lax.fori_loop((i + 1) // 16, ckn, j_body, 0)

            return 0

        cs = jnp.minimum(jnp.maximum(_sread(vstarts, cc) - s_al, 0), CAP)
        lax.fori_loop(cs, ce, i_body, 0)
        return 0

    lax.fori_loop(c_lo, c_hi, class_body, 0)

    def out_chunk(k, _):
        sl = pl.ds(k * 16, 16)
        keep = (vsup[sl] < 0.5) & (vsc[sl] > MIN_CONFIDENCE)
        vval[k // 8, pl.ds((k % 8) * 16, 16)] = jnp.where(keep, vsc[sl], 0.0)
        return 0

    lax.fori_loop(0, CAP // 16, out_chunk, 0)

    copies = [pltpu.async_copy(vval.at[r], outh.at[vidx.at[r]], sem)
              for r in range(NCHUNK)]
    for cp in copies:
        cp.wait()


_sc_nms = functools.partial(
    pl.kernel,
    out_type=jax.ShapeDtypeStruct((N + NW,), jnp.float32),
    mesh=plsc.VectorSubcoreMesh(core_axis_name="c", subcore_axis_name="s"),
    scratch_types=[
        pltpu.VMEM((CAPP,), jnp.float32),
        pltpu.VMEM((CAPP,), jnp.float32),
        pltpu.VMEM((CAPP,), jnp.float32),
        pltpu.VMEM((CAPP,), jnp.float32),
        pltpu.VMEM((CAPP,), jnp.float32),
        pltpu.VMEM((CAPP,), jnp.int32),
        pltpu.VMEM((CAPP,), jnp.float32),
        pltpu.VMEM((CAPP,), jnp.float32),
        pltpu.VMEM((NCHUNK, 128), jnp.int32),
        pltpu.VMEM((NCHUNK, 128), jnp.float32),
        pltpu.VMEM((STARTS_PAD,), jnp.int32),
        pltpu.VMEM((CB_PAD,), jnp.int32),
        pltpu.SemaphoreType.DMA,
    ],
)(_nms_body)


@jax.jit
def kernel(boxes, scores, labels):
    labels = labels.astype(jnp.int32)
    offs = labels.astype(jnp.float32)[:, None] * 4096.0
    b = boxes + offs
    idx = jnp.arange(N, dtype=jnp.int32)
    sl_, sneg, soi, sx1, sy1, sx2, sy2 = lax.sort(
        (labels, -scores, idx, b[:, 0], b[:, 1], b[:, 2], b[:, 3]), num_keys=3)
    ssc = -sneg

    starts = jnp.searchsorted(
        sl_, jnp.arange(NUM_CLASSES + 1, dtype=jnp.int32)).astype(jnp.int32)
    targets = jnp.arange(NW + 1, dtype=jnp.int32) * (N // NW)
    cb = jnp.searchsorted(starts, targets).astype(jnp.int32)

    pad = lambda a: jnp.pad(a, (0, PTOT - N))
    starts_p = jnp.pad(starts, (0, STARTS_PAD - (NUM_CLASSES + 1)))
    cb_p = jnp.pad(cb, (0, CB_PAD - (NW + 1)))

    out = _sc_nms(pad(sx1), pad(sy1), pad(sx2), pad(sy2), pad(ssc), pad(soi),
                  starts_p, cb_p)
    return out[:N]


# speedup vs baseline: 220.3240x; 18.8017x over previous
import functools

import jax
import jax.numpy as jnp
from jax import lax
from jax.experimental import pallas as pl
from jax.experimental.pallas import tpu as pltpu
from jax.experimental.pallas import tpu_sc as plsc

N = 20000
NUM_CLASSES = 1203
IOU_THRESHOLD = 0.5
MIN_CONFIDENCE = 0.2

NC = 2
NS = 16
NW = NC * NS
CAP = 2048
NCHUNK = CAP // 128
CAPP = CAP + 16
PTOT = N + CAP
DUMP = N
STARTS_PAD = 1232
CB_PAD = 48


def _sread(ref, i):
    return ref[pl.ds(i, 16)][0]


def _nms_body(x1h, y1h, x2h, y2h, sch, oih, sth, cbh, outh,
              vx1, vy1, vx2, vy2, vsc, voi, var, vsup, vidx, vval,
              vstarts, vcb, sem):
    c = lax.axis_index("c")
    s = lax.axis_index("s")
    w = s * NC + c

    pltpu.sync_copy(sth, vstarts)
    pltpu.sync_copy(cbh, vcb)

    c_lo = _sread(vcb, w)
    c_hi = _sread(vcb, w + 1)
    s_w = _sread(vstarts, c_lo)
    e_w = _sread(vstarts, c_hi)
    s_al = (s_w // 16) * 16
    e_loc = jnp.minimum(e_w - s_al, CAP)

    for hbm, vm in ((x1h, vx1), (y1h, vy1), (x2h, vx2), (y2h, vy2),
                    (sch, vsc), (oih, voi)):
        pltpu.sync_copy(hbm.at[pl.ds(s_al, CAP)], vm.at[pl.ds(0, CAP)])

    lane = lax.iota(jnp.int32, 16)

    def init_chunk(k, _):
        sl = pl.ds(k * 16, 16)
        x1 = vx1[sl]
        y1 = vy1[sl]
        x2 = vx2[sl]
        y2 = vy2[sl]
        var[sl] = (x2 - x1) * (y2 - y1)
        vsup[sl] = jnp.zeros((16,), jnp.float32)
        gpos = s_al + k * 16 + lane
        owned = (gpos >= s_w) & (gpos < e_w)
        midx = jnp.where(owned, voi[sl], DUMP + w * CAP + k * 16 + lane)
        vidx[k // 8, pl.ds((k % 8) * 16, 16)] = midx
        return 0

    lax.fori_loop(0, CAP // 16, init_chunk, 0)

    def class_body(cc, _):
        ce = jnp.minimum(_sread(vstarts, cc + 1) - s_al, CAP)
        ckn = (ce + 15) // 16

        def i_body(i, _):
            supi = _sread(vsup, i)
            sci = _sread(vsc, i)

            @pl.when((supi < 0.5) & (sci > MIN_CONFIDENCE))
            def _():
                bx1 = jnp.full((16,), _sread(vx1, i))
                by1 = jnp.full((16,), _sread(vy1, i))
                bx2 = jnp.full((16,), _sread(vx2, i))
                by2 = jnp.full((16,), _sread(vy2, i))
                ai = jnp.full((16,), _sread(var, i))

                def j_body(ck, _):
                    sl = pl.ds(ck * 16, 16)
                    jv = ck * 16 + lane
                    ix1 = jnp.maximum(bx1, vx1[sl])
                    iy1 = jnp.maximum(by1, vy1[sl])
                    ix2 = jnp.minimum(bx2, vx2[sl])
                    iy2 = jnp.minimum(by2, vy2[sl])
                    inter = jnp.maximum(ix2 - ix1, 0.0) * jnp.maximum(iy2 - iy1, 0.0)
                    iou = inter / (ai + var[sl] - inter + 1e-6)
                    hit = (iou > IOU_THRESHOLD) & (jv > i) & (jv < ce)
                    vsup[sl] = jnp.where(hit, 1.0, vsup[sl])
                    return 0

                lax.fori_loop((i + 1) // 16, ckn, j_body, 0)

            return 0

        cs = jnp.minimum(jnp.maximum(_sread(vstarts, cc) - s_al, 0), CAP)
        lax.fori_loop(cs, ce, i_body, 0)
        return 0

    lax.fori_loop(c_lo, c_hi, class_body, 0)

    def out_chunk(k, _):
        sl = pl.ds(k * 16, 16)
        keep = (vsup[sl] < 0.5) & (vsc[sl] > MIN_CONFIDENCE)
        vval[k // 8, pl.ds((k % 8) * 16, 16)] = jnp.where(keep, vsc[sl], 0.0)
        return 0

    lax.fori_loop(0, CAP // 16, out_chunk, 0)

    copies = [pltpu.async_copy(vval.at[r], outh.at[vidx.at[r]], sem)
              for r in range(NCHUNK)]
    for cp in copies:
        cp.wait()


_sc_nms = functools.partial(
    pl.kernel,
    out_type=jax.ShapeDtypeStruct((N + NW * CAP,), jnp.float32),
    mesh=plsc.VectorSubcoreMesh(core_axis_name="c", subcore_axis_name="s"),
    scratch_types=[
        pltpu.VMEM((CAPP,), jnp.float32),
        pltpu.VMEM((CAPP,), jnp.float32),
        pltpu.VMEM((CAPP,), jnp.float32),
        pltpu.VMEM((CAPP,), jnp.float32),
        pltpu.VMEM((CAPP,), jnp.float32),
        pltpu.VMEM((CAPP,), jnp.int32),
        pltpu.VMEM((CAPP,), jnp.float32),
        pltpu.VMEM((CAPP,), jnp.float32),
        pltpu.VMEM((NCHUNK, 128), jnp.int32),
        pltpu.VMEM((NCHUNK, 128), jnp.float32),
        pltpu.VMEM((STARTS_PAD,), jnp.int32),
        pltpu.VMEM((CB_PAD,), jnp.int32),
        pltpu.SemaphoreType.DMA,
    ],
)(_nms_body)


@jax.jit
def kernel(boxes, scores, labels):
    labels = labels.astype(jnp.int32)
    offs = labels.astype(jnp.float32)[:, None] * 4096.0
    b = boxes + offs
    idx = jnp.arange(N, dtype=jnp.int32)
    sl_, sneg, soi, sx1, sy1, sx2, sy2 = lax.sort(
        (labels, -scores, idx, b[:, 0], b[:, 1], b[:, 2], b[:, 3]), num_keys=3)
    ssc = -sneg

    starts = jnp.searchsorted(
        sl_, jnp.arange(NUM_CLASSES + 1, dtype=jnp.int32)).astype(jnp.int32)
    targets = jnp.arange(NW + 1, dtype=jnp.int32) * (N // NW)
    cb = jnp.searchsorted(starts, targets).astype(jnp.int32)

    pad = lambda a: jnp.pad(a, (0, PTOT - N))
    starts_p = jnp.pad(starts, (0, STARTS_PAD - (NUM_CLASSES + 1)))
    cb_p = jnp.pad(cb, (0, CB_PAD - (NW + 1)))

    out = _sc_nms(pad(sx1), pad(sy1), pad(sx2), pad(sy2), pad(ssc), pad(soi),
                  starts_p, cb_p)
    return out[:N]
